# baseline (speedup 1.0000x reference)
import functools
import math

import jax
import jax.numpy as jnp
from jax.experimental import pallas as pl

_B2A = 0.529177210903
_EV2H = 0.0367492929
_CUT = 0.06
_NRBF = 8
_NELEM = 10
_BI = 256
_BJ = 1024


def _bf(x):
    return x.astype(jnp.bfloat16).astype(jnp.float32)


def _pair_body(nj, bi, bj,
               pos_col_ref, pos_row_ref, z_col_ref, z_row_ref, z_col2_ref,
               zt_row_ref, zt_col_ref, we_ref, wr_ref, wro_ref,
               e_ref, g_ref):
    f32 = jnp.float32
    i = pl.program_id(0)
    pic = math.pi / _CUT

    wewro = we_ref[...] * wro_ref[...]
    c_t = jax.lax.dot_general(wr_ref[...], wewro,
                              (((1,), (1,)), ((), ())),
                              preferred_element_type=f32)

    zt_row = zt_row_ref[...]
    zt_col = zt_col_ref[...]

    ohi = (z_col_ref[...] == zt_row).astype(f32)
    ci = jax.lax.dot_general(ohi, c_t,
                             (((1,), (1,)), ((), ())),
                             preferred_element_type=f32)

    web = _bf(we_ref[...])
    wrob = _bf(wro_ref[...])
    wrbfb = _bf(wr_ref[...])
    hidden = web.shape[1]
    s_tabs = [jnp.zeros((bi, _NELEM), f32) for _ in range(_NRBF)]

    pos_i = pos_col_ref[...] * f32(_B2A)
    xi = pos_i[:, 0:1]
    yi = pos_i[:, 1:2]
    zi = pos_i[:, 2:3]

    gx = jnp.zeros((bi, 1), f32)
    gy = jnp.zeros((bi, 1), f32)
    gz = jnp.zeros((bi, 1), f32)

    for cidx in range(nj):
        j0 = cidx * bj
        pr = pos_row_ref[:, j0:j0 + bj] * f32(_B2A)
        xj = pr[0:1, :]
        yj = pr[1:2, :]
        zj = pr[2:3, :]
        zaj = z_row_ref[:, j0:j0 + bj]
        ohj = (zt_col == zaj).astype(f32)
        cj = jnp.dot(c_t, ohj, preferred_element_type=f32)
        ohj2 = (z_col2_ref[j0:j0 + bj, :] == zt_row).astype(f32)

        dx = xj - xi
        dy = yj - yi
        dz = zj - zi
        d2 = dx * dx + dy * dy + dz * dz
        mask = (d2 < f32(_CUT * _CUT)) & (d2 > f32(1e-16))
        d2s = jnp.where(mask, d2, f32(1.0))
        r = jnp.sqrt(d2s + f32(1e-18))
        rinv = f32(1.0) / r

        cinv = f32(1.0 / _CUT)
        q = jnp.clip(r * cinv, f32(0.0), f32(1.0))
        cose = jnp.cos(f32(math.pi) * q)
        env = f32(0.5) * (cose + f32(1.0))
        for k in range(1, _NRBF + 1):
            arg = (f32(k * math.pi) * r) * cinv
            rbf_k = (jnp.sin(arg) * rinv) * env
            rbfb_mk = jnp.where(mask, _bf(rbf_k), f32(0.0))
            s_tabs[k - 1] = s_tabs[k - 1] + jnp.dot(
                rbfb_mk, ohj2, preferred_element_type=f32)

        x = r * f32(pic)
        s1 = jnp.sin(x)
        c1 = jnp.cos(x)
        envg = f32(0.5) * (c1 + f32(1.0))
        two_c1 = c1 + c1
        sk, ck = s1, c1
        skm = jnp.zeros_like(s1)
        ckm = jnp.ones_like(c1)
        sg1 = jnp.zeros_like(s1)
        sg2 = jnp.zeros_like(s1)
        for k in range(1, _NRBF + 1):
            cik = ci[:, k - 1:k]
            cjk = cj[k - 1:k, :]
            wk = cik + cjk
            sg2 = sg2 + sk * wk
            sg1 = sg1 + f32(k) * (ck * wk)
            if k < _NRBF:
                sk, skm = two_c1 * sk - skm, sk
                ck, ckm = two_c1 * ck - ckm, ck

        er = envg * rinv
        denv_rinv = f32(-0.5) * f32(pic) * s1 * rinv
        gp = f32(pic) * er * sg1 + (denv_rinv - er * rinv) * sg2
        gf = jnp.where(mask, gp * rinv, f32(0.0))
        gx = gx - jnp.sum(gf * dx, axis=1, keepdims=True)
        gy = gy - jnp.sum(gf * dy, axis=1, keepdims=True)
        gz = gz - jnp.sum(gf * dz, axis=1, keepdims=True)

    g_ref[...] = jnp.concatenate([gx, gy, gz], axis=1) * f32(_EV2H * _B2A)

    agg = jnp.zeros((bi, hidden), f32)
    for k in range(_NRBF):
        for zz in range(_NELEM):
            v = web[zz:zz + 1, :] * wrbfb[k:k + 1, :]
            agg = agg + s_tabs[k][:, zz:zz + 1] * v
    node_e = jnp.sum(_bf(agg) * wrob, axis=1, keepdims=True)
    e_sum = jnp.sum(node_e)

    @pl.when(i == 0)
    def _init():
        e_ref[...] = jnp.zeros((1, 1), f32)

    e_ref[...] += (e_sum * f32(_EV2H)).reshape(1, 1)


def kernel(positions_bohr, atomic_numbers, z_table, W_embed, W_rbf, w_readout):
    nat = positions_bohr.shape[0]
    nelem = z_table.shape[0]
    hidden = W_embed.shape[1]
    bi = min(_BI, nat)
    bj = min(_BJ, nat)
    ni = nat // bi
    nj = nat // bj

    pos_col = positions_bohr.astype(jnp.float32)
    pos_row = pos_col.T
    z_col = atomic_numbers.reshape(nat, 1)
    z_row = atomic_numbers.reshape(1, nat)
    zt_row = z_table.reshape(1, nelem)
    zt_col = z_table.reshape(nelem, 1)
    wro = w_readout.reshape(1, hidden)

    body = functools.partial(_pair_body, nj, bi, bj)
    e, g = pl.pallas_call(
        body,
        grid=(ni,),
        in_specs=[
            pl.BlockSpec((bi, 3), lambda i: (i, 0)),
            pl.BlockSpec((3, nat), lambda i: (0, 0)),
            pl.BlockSpec((bi, 1), lambda i: (i, 0)),
            pl.BlockSpec((1, nat), lambda i: (0, 0)),
            pl.BlockSpec((nat, 1), lambda i: (0, 0)),
            pl.BlockSpec((1, nelem), lambda i: (0, 0)),
            pl.BlockSpec((nelem, 1), lambda i: (0, 0)),
            pl.BlockSpec((nelem, hidden), lambda i: (0, 0)),
            pl.BlockSpec((_NRBF, hidden), lambda i: (0, 0)),
            pl.BlockSpec((1, hidden), lambda i: (0, 0)),
        ],
        out_specs=[
            pl.BlockSpec((1, 1), lambda i: (0, 0)),
            pl.BlockSpec((bi, 3), lambda i: (i, 0)),
        ],
        out_shape=[
            jax.ShapeDtypeStruct((1, 1), jnp.float32),
            jax.ShapeDtypeStruct((nat, 3), jnp.float32),
        ],
    )(pos_col, pos_row, z_col, z_row, z_col, zt_row, zt_col,
      W_embed, W_rbf, wro)
    return e.reshape(1), g


# speedup vs baseline: 1.0310x; 1.0310x over previous
import functools
import math

import jax
import jax.numpy as jnp
from jax.experimental import pallas as pl

_B2A = 0.529177210903
_EV2H = 0.0367492929
_CUT = 0.06
_NRBF = 8
_NELEM = 10
_BI = 512
_BJ = 1024


def _bf(x):
    return x.astype(jnp.bfloat16).astype(jnp.float32)


def _pair_body(nj, bi, bj,
               pos_col_ref, pos_row_ref, z_col_ref, z_row_ref, z_col2_ref,
               zt_row_ref, zt_col_ref, we_ref, wr_ref, wro_ref,
               e_ref, g_ref):
    f32 = jnp.float32
    i = pl.program_id(0)
    pic = math.pi / _CUT

    wewro = we_ref[...] * wro_ref[...]
    c_t = jax.lax.dot_general(wr_ref[...], wewro,
                              (((1,), (1,)), ((), ())),
                              preferred_element_type=f32)

    zt_row = zt_row_ref[...]
    zt_col = zt_col_ref[...]

    ohi = (z_col_ref[...] == zt_row).astype(f32)
    ci = jax.lax.dot_general(ohi, c_t,
                             (((1,), (1,)), ((), ())),
                             preferred_element_type=f32)

    web = _bf(we_ref[...])
    wrob = _bf(wro_ref[...])
    wrbfb = _bf(wr_ref[...])
    hidden = web.shape[1]
    s_tabs = [jnp.zeros((bi, _NELEM), f32) for _ in range(_NRBF)]

    pos_i = pos_col_ref[...] * f32(_B2A)
    xi = pos_i[:, 0:1]
    yi = pos_i[:, 1:2]
    zi = pos_i[:, 2:3]

    gx = jnp.zeros((bi, 1), f32)
    gy = jnp.zeros((bi, 1), f32)
    gz = jnp.zeros((bi, 1), f32)

    for cidx in range(nj):
        j0 = cidx * bj
        pr = pos_row_ref[:, j0:j0 + bj] * f32(_B2A)
        xj = pr[0:1, :]
        yj = pr[1:2, :]
        zj = pr[2:3, :]
        zaj = z_row_ref[:, j0:j0 + bj]
        ohj = (zt_col == zaj).astype(f32)
        cj = jnp.dot(c_t, ohj, preferred_element_type=f32)
        ohj2 = (z_col2_ref[j0:j0 + bj, :] == zt_row).astype(f32)

        dx = xj - xi
        dy = yj - yi
        dz = zj - zi
        d2 = dx * dx + dy * dy + dz * dz
        mask = (d2 < f32(_CUT * _CUT)) & (d2 > f32(1e-16))
        d2s = jnp.where(mask, d2, f32(1.0))
        r = jnp.sqrt(d2s + f32(1e-18))
        rinv = f32(1.0) / r

        q = jnp.clip(r / f32(_CUT), f32(0.0), f32(1.0))
        cose = jnp.cos(f32(math.pi) * q)
        env = f32(0.5) * (cose + f32(1.0))
        for k in range(1, _NRBF + 1):
            arg = (f32(k * math.pi) * r) / f32(_CUT)
            rbf_k = (jnp.sin(arg) / r) * env
            rbfb_mk = jnp.where(mask, _bf(rbf_k), f32(0.0))
            s_tabs[k - 1] = s_tabs[k - 1] + jnp.dot(
                rbfb_mk, ohj2, preferred_element_type=f32)

        x = r * f32(pic)
        s1 = jnp.sin(x)
        c1 = jnp.cos(x)
        envg = f32(0.5) * (c1 + f32(1.0))
        two_c1 = c1 + c1
        sk, ck = s1, c1
        skm = jnp.zeros_like(s1)
        ckm = jnp.ones_like(c1)
        sg1 = jnp.zeros_like(s1)
        sg2 = jnp.zeros_like(s1)
        for k in range(1, _NRBF + 1):
            cik = ci[:, k - 1:k]
            cjk = cj[k - 1:k, :]
            wk = cik + cjk
            sg2 = sg2 + sk * wk
            sg1 = sg1 + f32(k) * (ck * wk)
            if k < _NRBF:
                sk, skm = two_c1 * sk - skm, sk
                ck, ckm = two_c1 * ck - ckm, ck

        er = envg * rinv
        denv_rinv = f32(-0.5) * f32(pic) * s1 * rinv
        gp = f32(pic) * er * sg1 + (denv_rinv - er * rinv) * sg2
        gf = jnp.where(mask, gp * rinv, f32(0.0))
        gx = gx - jnp.sum(gf * dx, axis=1, keepdims=True)
        gy = gy - jnp.sum(gf * dy, axis=1, keepdims=True)
        gz = gz - jnp.sum(gf * dz, axis=1, keepdims=True)

    g_ref[...] = jnp.concatenate([gx, gy, gz], axis=1) * f32(_EV2H * _B2A)

    agg = jnp.zeros((bi, hidden), f32)
    for k in range(_NRBF):
        for zz in range(_NELEM):
            v = web[zz:zz + 1, :] * wrbfb[k:k + 1, :]
            agg = agg + s_tabs[k][:, zz:zz + 1] * v
    node_e = jnp.sum(_bf(agg) * wrob, axis=1, keepdims=True)
    e_sum = jnp.sum(node_e)

    @pl.when(i == 0)
    def _init():
        e_ref[...] = jnp.zeros((1, 1), f32)

    e_ref[...] += (e_sum * f32(_EV2H)).reshape(1, 1)


def kernel(positions_bohr, atomic_numbers, z_table, W_embed, W_rbf, w_readout):
    nat = positions_bohr.shape[0]
    nelem = z_table.shape[0]
    hidden = W_embed.shape[1]
    bi = min(_BI, nat)
    bj = min(_BJ, nat)
    ni = nat // bi
    nj = nat // bj

    pos_col = positions_bohr.astype(jnp.float32)
    pos_row = pos_col.T
    z_col = atomic_numbers.reshape(nat, 1)
    z_row = atomic_numbers.reshape(1, nat)
    zt_row = z_table.reshape(1, nelem)
    zt_col = z_table.reshape(nelem, 1)
    wro = w_readout.reshape(1, hidden)

    body = functools.partial(_pair_body, nj, bi, bj)
    e, g = pl.pallas_call(
        body,
        grid=(ni,),
        in_specs=[
            pl.BlockSpec((bi, 3), lambda i: (i, 0)),
            pl.BlockSpec((3, nat), lambda i: (0, 0)),
            pl.BlockSpec((bi, 1), lambda i: (i, 0)),
            pl.BlockSpec((1, nat), lambda i: (0, 0)),
            pl.BlockSpec((nat, 1), lambda i: (0, 0)),
            pl.BlockSpec((1, nelem), lambda i: (0, 0)),
            pl.BlockSpec((nelem, 1), lambda i: (0, 0)),
            pl.BlockSpec((nelem, hidden), lambda i: (0, 0)),
            pl.BlockSpec((_NRBF, hidden), lambda i: (0, 0)),
            pl.BlockSpec((1, hidden), lambda i: (0, 0)),
        ],
        out_specs=[
            pl.BlockSpec((1, 1), lambda i: (0, 0)),
            pl.BlockSpec((bi, 3), lambda i: (i, 0)),
        ],
        out_shape=[
            jax.ShapeDtypeStruct((1, 1), jnp.float32),
            jax.ShapeDtypeStruct((nat, 3), jnp.float32),
        ],
    )(pos_col, pos_row, z_col, z_row, z_col, zt_row, zt_col,
      W_embed, W_rbf, wro)
    return e.reshape(1), g
